# baseline (speedup 1.0000x reference)
import functools

import jax
import jax.numpy as jnp
from jax.experimental import pallas as pl
from jax.experimental.pallas import tpu as pltpu

LEAKY_ALPHA = 0.2


def _proj_kernel(x_ref, w_ref, asrc_ref, adst_ref, h_ref, ss_ref, sd_ref):
    x = x_ref[0]
    w = w_ref[0]
    h = jnp.dot(x, w, preferred_element_type=jnp.float32)
    t = jnp.tanh(h)
    ss = jax.lax.dot_general(t, asrc_ref[0], (((1,), (0,)), ((), ())),
                             preferred_element_type=jnp.float32)
    sd = jax.lax.dot_general(adst_ref[0], t, (((0,), (1,)), ((), ())),
                             preferred_element_type=jnp.float32)
    h_ref[0, 0] = h
    ss_ref[0, 0] = ss
    sd_ref[0, 0] = sd


def _softmax_rows(ss, sd):
    m = ss + jnp.max(sd)
    m = jnp.where(m > 0, m, LEAKY_ALPHA * m)
    p = ss - m
    q = LEAKY_ALPHA * ss - m
    return jnp.exp(jnp.maximum(p + sd, q + LEAKY_ALPHA * sd))


def _normalize(e):
    denom = jnp.sum(e, axis=1, keepdims=True)
    zero_row = denom == 0.0
    r = 1.0 / jnp.where(zero_row, 1.0, denom)
    u = jnp.where(zero_row, 1.0 / e.shape[1], 0.0)
    return e * r + u


def _attn0_kernel(ss_ref, sd_ref, adj_ref, h_ref, b_ref,
                  w2_ref, as2_ref, ad2_ref,
                  adji_ref, h2_ref, ss2_ref, sd2_ref,
                  *, n_heads, d_out, n_heads2):
    adj = adj_ref[0]
    cols = []
    for head in range(n_heads):
        e = _softmax_rows(ss_ref[0, head], sd_ref[0, head]) * adj
        a = _normalize(e)
        cols.append(jnp.dot(a, h_ref[0, head],
                            preferred_element_type=jnp.float32) + b_ref[0])
    x1 = jnp.concatenate(cols, axis=1)
    adji_ref[0] = adj.astype(jnp.int8)
    for oh in range(n_heads2):
        hh = jnp.dot(x1, w2_ref[oh], preferred_element_type=jnp.float32)
        t = jnp.tanh(hh)
        h2_ref[0, oh] = hh
        ss2_ref[0, oh] = jax.lax.dot_general(
            t, as2_ref[oh], (((1,), (0,)), ((), ())),
            preferred_element_type=jnp.float32)
        sd2_ref[0, oh] = jax.lax.dot_general(
            ad2_ref[oh], t, (((0,), (1,)), ((), ())),
            preferred_element_type=jnp.float32)


def _attn1_kernel(ss_ref, sd_ref, adji_ref, h_ref, b_ref, x_ref, w_ref,
                  *, n_heads, d_out):
    masked = adji_ref[0] == 0
    for head in range(n_heads):
        e = _softmax_rows(ss_ref[0, head], sd_ref[0, head])
        e = jnp.where(masked, 0.0, e)
        a = _normalize(e)
        x_ref[0, :, head * d_out:(head + 1) * d_out] = (
            jnp.dot(a, h_ref[0, head],
                    preferred_element_type=jnp.float32) + b_ref[0])
        w_ref[0, head] = a


def kernel(node_feature, adj, w0, a_src0, a_dst0, b0, w1, a_src1, a_dst1, b1):
    B, N, in_dim = node_feature.shape
    H, _, D = w0.shape
    H2, in2, D2 = w1.shape

    h0, ss0, sd0 = pl.pallas_call(
        _proj_kernel,
        grid=(B, H),
        in_specs=[
            pl.BlockSpec((1, N, in_dim), lambda bb, hh: (bb, 0, 0)),
            pl.BlockSpec((1, in_dim, D), lambda bb, hh: (hh, 0, 0)),
            pl.BlockSpec((1, D, 1), lambda bb, hh: (hh, 0, 0)),
            pl.BlockSpec((1, D, 1), lambda bb, hh: (hh, 0, 0)),
        ],
        out_specs=[
            pl.BlockSpec((1, 1, N, D), lambda bb, hh: (bb, hh, 0, 0)),
            pl.BlockSpec((1, 1, N, 1), lambda bb, hh: (bb, hh, 0, 0)),
            pl.BlockSpec((1, 1, 1, N), lambda bb, hh: (bb, hh, 0, 0)),
        ],
        out_shape=[
            jax.ShapeDtypeStruct((B, H, N, D), jnp.float32),
            jax.ShapeDtypeStruct((B, H, N, 1), jnp.float32),
            jax.ShapeDtypeStruct((B, H, 1, N), jnp.float32),
        ],
    )(node_feature, w0, a_src0, a_dst0)

    BK0 = 512
    R0 = N // BK0
    adj_i8, h1, ss1, sd1 = pl.pallas_call(
        functools.partial(_attn0_kernel, n_heads=H, d_out=D, n_heads2=H2),
        grid=(B, R0),
        in_specs=[
            pl.BlockSpec((1, H, BK0, 1), lambda bb, rr: (bb, 0, rr, 0)),
            pl.BlockSpec((1, H, 1, N), lambda bb, rr: (bb, 0, 0, 0)),
            pl.BlockSpec((1, BK0, N), lambda bb, rr: (bb, rr, 0)),
            pl.BlockSpec((1, H, N, D), lambda bb, rr: (bb, 0, 0, 0)),
            pl.BlockSpec((1, D), lambda bb, rr: (0, 0)),
            pl.BlockSpec((H2, in2, D2), lambda bb, rr: (0, 0, 0)),
            pl.BlockSpec((H2, D2, 1), lambda bb, rr: (0, 0, 0)),
            pl.BlockSpec((H2, D2, 1), lambda bb, rr: (0, 0, 0)),
        ],
        out_specs=[
            pl.BlockSpec((1, BK0, N), lambda bb, rr: (bb, rr, 0)),
            pl.BlockSpec((1, H2, BK0, D2), lambda bb, rr: (bb, 0, rr, 0)),
            pl.BlockSpec((1, H2, BK0, 1), lambda bb, rr: (bb, 0, rr, 0)),
            pl.BlockSpec((1, H2, 1, BK0), lambda bb, rr: (bb, 0, 0, rr)),
        ],
        out_shape=[
            jax.ShapeDtypeStruct((B, N, N), jnp.int8),
            jax.ShapeDtypeStruct((B, H2, N, D2), jnp.float32),
            jax.ShapeDtypeStruct((B, H2, N, 1), jnp.float32),
            jax.ShapeDtypeStruct((B, H2, 1, N), jnp.float32),
        ],
    )(ss0, sd0, adj, h0, b0.reshape(1, D), w1, a_src1, a_dst1)

    BK1 = 256
    R1 = N // BK1
    x_out, weight = pl.pallas_call(
        functools.partial(_attn1_kernel, n_heads=H2, d_out=D2),
        grid=(B, R1),
        in_specs=[
            pl.BlockSpec((1, H2, BK1, 1), lambda bb, rr: (bb, 0, rr, 0)),
            pl.BlockSpec((1, H2, 1, N), lambda bb, rr: (bb, 0, 0, 0)),
            pl.BlockSpec((1, BK1, N), lambda bb, rr: (bb, rr, 0)),
            pl.BlockSpec((1, H2, N, D2), lambda bb, rr: (bb, 0, 0, 0)),
            pl.BlockSpec((1, D2), lambda bb, rr: (0, 0)),
        ],
        out_specs=[
            pl.BlockSpec((1, BK1, H2 * D2), lambda bb, rr: (bb, rr, 0)),
            pl.BlockSpec((1, H2, BK1, N), lambda bb, rr: (bb, 0, rr, 0)),
        ],
        out_shape=[
            jax.ShapeDtypeStruct((B, N, H2 * D2), jnp.float32),
            jax.ShapeDtypeStruct((B, H2, N, N), jnp.float32),
        ],
    )(ss1, sd1, adj_i8, h1, b1.reshape(1, D2))

    return x_out, weight


# speedup vs baseline: 1.2200x; 1.2200x over previous
import functools

import jax
import jax.numpy as jnp
from jax.experimental import pallas as pl
from jax.experimental.pallas import tpu as pltpu

LEAKY_ALPHA = 0.2


def _proj_kernel(x_ref, w_ref, asrc_ref, adst_ref, h_ref, ss_ref, sd_ref):
    x = x_ref[0]
    w = w_ref[0]
    h = jnp.dot(x, w, preferred_element_type=jnp.float32)
    t = jnp.tanh(h)
    ss = jax.lax.dot_general(t, asrc_ref[0], (((1,), (0,)), ((), ())),
                             preferred_element_type=jnp.float32)
    sd = jax.lax.dot_general(adst_ref[0], t, (((0,), (1,)), ((), ())),
                             preferred_element_type=jnp.float32)
    h_ref[0, 0] = h
    ss_ref[0, 0] = ss
    sd_ref[0, 0] = sd


def _exp_factors(ss, sd):
    M = jnp.max(sd)
    z = ss + M
    ep = jnp.exp(0.8 * jnp.minimum(z, 0.0))
    eq = jnp.exp(-0.8 * jnp.maximum(z, 0.0))
    esd = jnp.exp(sd - M)
    esd2 = jnp.exp(LEAKY_ALPHA * (sd - M))
    return ep, eq, esd, esd2


def _ones_col(h):
    return jnp.concatenate([h, jnp.ones((h.shape[0], 1), h.dtype)], axis=1)


def _norm_cols(y, d, n):
    denom = y[:, d:d + 1]
    zero_row = denom == 0.0
    r = 1.0 / jnp.where(zero_row, 1.0, denom)
    u = jnp.where(zero_row, 1.0 / n, 0.0)
    return r, u, zero_row


def _attn0_kernel(ss_ref, sd_ref, adj_ref, h_ref, b_ref,
                  w2_ref, as2_ref, ad2_ref,
                  adji_ref, h2_ref, ss2_ref, sd2_ref,
                  *, n_heads, d_out, n_heads2):
    adj = adj_ref[0]
    n = adj.shape[1]
    cols = []
    for head in range(n_heads):
        ep, eq, esd, esd2 = _exp_factors(ss_ref[0, head], sd_ref[0, head])
        e = jnp.maximum(ep * esd, eq * esd2) * adj
        h = h_ref[0, head]
        y = jnp.dot(e, _ones_col(h), preferred_element_type=jnp.float32)
        r, u, zero_row = _norm_cols(y, d_out, n)
        hm = jnp.sum(h, axis=0, keepdims=True) / n
        xh = jnp.where(zero_row, hm, y[:, :d_out] * r)
        cols.append(xh + b_ref[0])
    x1 = jnp.concatenate(cols, axis=1)
    adji_ref[0] = adj.astype(jnp.int8)
    for oh in range(n_heads2):
        hh = jnp.dot(x1, w2_ref[oh], preferred_element_type=jnp.float32)
        t = jnp.tanh(hh)
        h2_ref[0, oh] = hh
        ss2_ref[0, oh] = jax.lax.dot_general(
            t, as2_ref[oh], (((1,), (0,)), ((), ())),
            preferred_element_type=jnp.float32)
        sd2_ref[0, oh] = jax.lax.dot_general(
            ad2_ref[oh], t, (((0,), (1,)), ((), ())),
            preferred_element_type=jnp.float32)


def _attn1_kernel(ss_ref, sd_ref, adji_ref, h_ref, b_ref, x_ref, w_ref,
                  *, n_heads, d_out):
    masked = adji_ref[0] == 0
    n = masked.shape[1]
    for head in range(n_heads):
        ep, eq, esd, esd2 = _exp_factors(ss_ref[0, head], sd_ref[0, head])
        e = jnp.where(masked, 0.0, jnp.maximum(ep * esd, eq * esd2))
        h = h_ref[0, head]
        y = jnp.dot(e, _ones_col(h), preferred_element_type=jnp.float32)
        r, u, zero_row = _norm_cols(y, d_out, n)
        w_ref[0, head] = e * r + u
        hm = jnp.sum(h, axis=0, keepdims=True) / n
        xh = jnp.where(zero_row, hm, y[:, :d_out] * r)
        x_ref[0, :, head * d_out:(head + 1) * d_out] = xh + b_ref[0]


def kernel(node_feature, adj, w0, a_src0, a_dst0, b0, w1, a_src1, a_dst1, b1):
    B, N, in_dim = node_feature.shape
    H, _, D = w0.shape
    H2, in2, D2 = w1.shape

    h0, ss0, sd0 = pl.pallas_call(
        _proj_kernel,
        grid=(B, H),
        in_specs=[
            pl.BlockSpec((1, N, in_dim), lambda bb, hh: (bb, 0, 0)),
            pl.BlockSpec((1, in_dim, D), lambda bb, hh: (hh, 0, 0)),
            pl.BlockSpec((1, D, 1), lambda bb, hh: (hh, 0, 0)),
            pl.BlockSpec((1, D, 1), lambda bb, hh: (hh, 0, 0)),
        ],
        out_specs=[
            pl.BlockSpec((1, 1, N, D), lambda bb, hh: (bb, hh, 0, 0)),
            pl.BlockSpec((1, 1, N, 1), lambda bb, hh: (bb, hh, 0, 0)),
            pl.BlockSpec((1, 1, 1, N), lambda bb, hh: (bb, hh, 0, 0)),
        ],
        out_shape=[
            jax.ShapeDtypeStruct((B, H, N, D), jnp.float32),
            jax.ShapeDtypeStruct((B, H, N, 1), jnp.float32),
            jax.ShapeDtypeStruct((B, H, 1, N), jnp.float32),
        ],
    )(node_feature, w0, a_src0, a_dst0)

    BK0 = 512
    R0 = N // BK0
    adj_i8, h1, ss1, sd1 = pl.pallas_call(
        functools.partial(_attn0_kernel, n_heads=H, d_out=D, n_heads2=H2),
        grid=(B, R0),
        in_specs=[
            pl.BlockSpec((1, H, BK0, 1), lambda bb, rr: (bb, 0, rr, 0)),
            pl.BlockSpec((1, H, 1, N), lambda bb, rr: (bb, 0, 0, 0)),
            pl.BlockSpec((1, BK0, N), lambda bb, rr: (bb, rr, 0)),
            pl.BlockSpec((1, H, N, D), lambda bb, rr: (bb, 0, 0, 0)),
            pl.BlockSpec((1, D), lambda bb, rr: (0, 0)),
            pl.BlockSpec((H2, in2, D2), lambda bb, rr: (0, 0, 0)),
            pl.BlockSpec((H2, D2, 1), lambda bb, rr: (0, 0, 0)),
            pl.BlockSpec((H2, D2, 1), lambda bb, rr: (0, 0, 0)),
        ],
        out_specs=[
            pl.BlockSpec((1, BK0, N), lambda bb, rr: (bb, rr, 0)),
            pl.BlockSpec((1, H2, BK0, D2), lambda bb, rr: (bb, 0, rr, 0)),
            pl.BlockSpec((1, H2, BK0, 1), lambda bb, rr: (bb, 0, rr, 0)),
            pl.BlockSpec((1, H2, 1, BK0), lambda bb, rr: (bb, 0, 0, rr)),
        ],
        out_shape=[
            jax.ShapeDtypeStruct((B, N, N), jnp.int8),
            jax.ShapeDtypeStruct((B, H2, N, D2), jnp.float32),
            jax.ShapeDtypeStruct((B, H2, N, 1), jnp.float32),
            jax.ShapeDtypeStruct((B, H2, 1, N), jnp.float32),
        ],
    )(ss0, sd0, adj, h0, b0.reshape(1, D), w1, a_src1, a_dst1)

    BK1 = 256
    R1 = N // BK1
    x_out, weight = pl.pallas_call(
        functools.partial(_attn1_kernel, n_heads=H2, d_out=D2),
        grid=(B, R1),
        in_specs=[
            pl.BlockSpec((1, H2, BK1, 1), lambda bb, rr: (bb, 0, rr, 0)),
            pl.BlockSpec((1, H2, 1, N), lambda bb, rr: (bb, 0, 0, 0)),
            pl.BlockSpec((1, BK1, N), lambda bb, rr: (bb, rr, 0)),
            pl.BlockSpec((1, H2, N, D2), lambda bb, rr: (bb, 0, 0, 0)),
            pl.BlockSpec((1, D2), lambda bb, rr: (0, 0)),
        ],
        out_specs=[
            pl.BlockSpec((1, BK1, H2 * D2), lambda bb, rr: (bb, rr, 0)),
            pl.BlockSpec((1, H2, BK1, N), lambda bb, rr: (bb, 0, rr, 0)),
        ],
        out_shape=[
            jax.ShapeDtypeStruct((B, N, H2 * D2), jnp.float32),
            jax.ShapeDtypeStruct((B, H2, N, N), jnp.float32),
        ],
    )(ss1, sd1, adj_i8, h1, b1.reshape(1, D2))

    return x_out, weight


# speedup vs baseline: 1.2866x; 1.0546x over previous
import functools

import jax
import jax.numpy as jnp
from jax.experimental import pallas as pl
from jax.experimental.pallas import tpu as pltpu

LEAKY_ALPHA = 0.2


def _exp_factors(ss, sd):
    M = jnp.max(sd)
    z = ss + M
    ep = jnp.exp(0.8 * jnp.minimum(z, 0.0))
    eq = jnp.exp(-0.8 * jnp.maximum(z, 0.0))
    esd = jnp.exp(sd - M)
    esd2 = jnp.exp(LEAKY_ALPHA * (sd - M))
    return ep, eq, esd, esd2


def _ones_col(h):
    return jnp.concatenate([h, jnp.ones((h.shape[0], 1), h.dtype)], axis=1)


def _norm_cols(y, d, n):
    denom = y[:, d:d + 1]
    zero_row = denom == 0.0
    r = 1.0 / jnp.where(zero_row, 1.0, denom)
    u = jnp.where(zero_row, 1.0 / n, 0.0)
    return r, u, zero_row


def _project(x, w_ref, asrc_ref, adst_ref, h_ref, ss_ref, sd_ref, n_heads):
    for head in range(n_heads):
        hh = jnp.dot(x, w_ref[head], preferred_element_type=jnp.float32)
        t = jnp.tanh(hh)
        h_ref[head] = hh
        ss_ref[head] = jax.lax.dot_general(
            t, asrc_ref[head], (((1,), (0,)), ((), ())),
            preferred_element_type=jnp.float32)
        sd_ref[head] = jax.lax.dot_general(
            adst_ref[head], t, (((0,), (1,)), ((), ())),
            preferred_element_type=jnp.float32)


def _layer0_kernel(x_ref, w0_ref, as0_ref, ad0_ref, adj_ref, b_ref,
                   w2_ref, as2_ref, ad2_ref,
                   adji_ref, h2_ref, ss2_ref, sd2_ref,
                   h0s, ss0s, sd0s,
                   *, n_heads, d_out, n_heads2, block_rows):
    rr = pl.program_id(1)

    @pl.when(rr == 0)
    def _proj_phase():
        _project(x_ref[0], w0_ref, as0_ref, ad0_ref, h0s, ss0s, sd0s, n_heads)

    @pl.when(rr > 0)
    def _attn_phase():
        adj = adj_ref[0]
        n = adj.shape[1]
        row0 = (rr - 1) * block_rows
        cols = []
        for head in range(n_heads):
            ep, eq, esd, esd2 = _exp_factors(
                ss0s[head, pl.ds(row0, block_rows)], sd0s[head])
            e = jnp.maximum(ep * esd, eq * esd2) * adj
            h = h0s[head]
            y = jnp.dot(e, _ones_col(h), preferred_element_type=jnp.float32)
            r, u, zero_row = _norm_cols(y, d_out, n)
            hm = jnp.sum(h, axis=0, keepdims=True) / n
            xh = jnp.where(zero_row, hm, y[:, :d_out] * r)
            cols.append(xh + b_ref[0])
        x1 = jnp.concatenate(cols, axis=1)
        adji_ref[0] = adj.astype(jnp.int8)
        for oh in range(n_heads2):
            hh = jnp.dot(x1, w2_ref[oh], preferred_element_type=jnp.float32)
            t = jnp.tanh(hh)
            h2_ref[0, oh] = hh
            ss2_ref[0, oh] = jax.lax.dot_general(
                t, as2_ref[oh], (((1,), (0,)), ((), ())),
                preferred_element_type=jnp.float32)
            sd2_ref[0, oh] = jax.lax.dot_general(
                ad2_ref[oh], t, (((0,), (1,)), ((), ())),
                preferred_element_type=jnp.float32)


def _attn1_kernel(ss_ref, sd_ref, adji_ref, h_ref, b_ref, x_ref, w_ref,
                  *, n_heads, d_out):
    masked = adji_ref[0] == 0
    n = masked.shape[1]
    for head in range(n_heads):
        ep, eq, esd, esd2 = _exp_factors(ss_ref[0, head], sd_ref[0, head])
        e = jnp.where(masked, 0.0, jnp.maximum(ep * esd, eq * esd2))
        h = h_ref[0, head]
        y = jnp.dot(e, _ones_col(h), preferred_element_type=jnp.float32)
        r, u, zero_row = _norm_cols(y, d_out, n)
        w_ref[0, head] = e * r + u
        hm = jnp.sum(h, axis=0, keepdims=True) / n
        xh = jnp.where(zero_row, hm, y[:, :d_out] * r)
        x_ref[0, :, head * d_out:(head + 1) * d_out] = xh + b_ref[0]


def kernel(node_feature, adj, w0, a_src0, a_dst0, b0, w1, a_src1, a_dst1, b1):
    B, N, in_dim = node_feature.shape
    H, _, D = w0.shape
    H2, in2, D2 = w1.shape

    BK0 = 512
    R0 = N // BK0
    prev = lambda rr: jnp.maximum(rr - 1, 0)
    adj_i8, h1, ss1, sd1 = pl.pallas_call(
        functools.partial(_layer0_kernel, n_heads=H, d_out=D, n_heads2=H2,
                          block_rows=BK0),
        grid=(B, R0 + 1),
        in_specs=[
            pl.BlockSpec((1, N, in_dim), lambda bb, rr: (bb, 0, 0)),
            pl.BlockSpec((H, in_dim, D), lambda bb, rr: (0, 0, 0)),
            pl.BlockSpec((H, D, 1), lambda bb, rr: (0, 0, 0)),
            pl.BlockSpec((H, D, 1), lambda bb, rr: (0, 0, 0)),
            pl.BlockSpec((1, BK0, N), lambda bb, rr: (bb, prev(rr), 0)),
            pl.BlockSpec((1, D), lambda bb, rr: (0, 0)),
            pl.BlockSpec((H2, in2, D2), lambda bb, rr: (0, 0, 0)),
            pl.BlockSpec((H2, D2, 1), lambda bb, rr: (0, 0, 0)),
            pl.BlockSpec((H2, D2, 1), lambda bb, rr: (0, 0, 0)),
        ],
        out_specs=[
            pl.BlockSpec((1, BK0, N), lambda bb, rr: (bb, prev(rr), 0)),
            pl.BlockSpec((1, H2, BK0, D2), lambda bb, rr: (bb, 0, prev(rr), 0)),
            pl.BlockSpec((1, H2, BK0, 1), lambda bb, rr: (bb, 0, prev(rr), 0)),
            pl.BlockSpec((1, H2, 1, BK0), lambda bb, rr: (bb, 0, 0, prev(rr))),
        ],
        out_shape=[
            jax.ShapeDtypeStruct((B, N, N), jnp.int8),
            jax.ShapeDtypeStruct((B, H2, N, D2), jnp.float32),
            jax.ShapeDtypeStruct((B, H2, N, 1), jnp.float32),
            jax.ShapeDtypeStruct((B, H2, 1, N), jnp.float32),
        ],
        scratch_shapes=[
            pltpu.VMEM((H, N, D), jnp.float32),
            pltpu.VMEM((H, N, 1), jnp.float32),
            pltpu.VMEM((H, 1, N), jnp.float32),
        ],
    )(node_feature, w0, a_src0, a_dst0, adj, b0.reshape(1, D),
      w1, a_src1, a_dst1)

    BK1 = 256
    R1 = N // BK1
    x_out, weight = pl.pallas_call(
        functools.partial(_attn1_kernel, n_heads=H2, d_out=D2),
        grid=(B, R1),
        in_specs=[
            pl.BlockSpec((1, H2, BK1, 1), lambda bb, rr: (bb, 0, rr, 0)),
            pl.BlockSpec((1, H2, 1, N), lambda bb, rr: (bb, 0, 0, 0)),
            pl.BlockSpec((1, BK1, N), lambda bb, rr: (bb, rr, 0)),
            pl.BlockSpec((1, H2, N, D2), lambda bb, rr: (bb, 0, 0, 0)),
            pl.BlockSpec((1, D2), lambda bb, rr: (0, 0)),
        ],
        out_specs=[
            pl.BlockSpec((1, BK1, H2 * D2), lambda bb, rr: (bb, rr, 0)),
            pl.BlockSpec((1, H2, BK1, N), lambda bb, rr: (bb, 0, rr, 0)),
        ],
        out_shape=[
            jax.ShapeDtypeStruct((B, N, H2 * D2), jnp.float32),
            jax.ShapeDtypeStruct((B, H2, N, N), jnp.float32),
        ],
    )(ss1, sd1, adj_i8, h1, b1.reshape(1, D2))

    return x_out, weight


# speedup vs baseline: 1.2881x; 1.0012x over previous
import functools

import jax
import jax.numpy as jnp
from jax.experimental import pallas as pl
from jax.experimental.pallas import tpu as pltpu

LEAKY_ALPHA = 0.2


def _exp_factors(ss, sd):
    M = jnp.max(sd)
    z = ss + M
    ep = jnp.exp(0.8 * jnp.minimum(z, 0.0))
    eq = jnp.exp(-0.8 * jnp.maximum(z, 0.0))
    esd = jnp.exp(sd - M)
    esd2 = jnp.exp(LEAKY_ALPHA * (sd - M))
    return ep, eq, esd, esd2


def _ones_col(h):
    return jnp.concatenate([h, jnp.ones((h.shape[0], 1), h.dtype)], axis=1)


def _norm_cols(y, d, n):
    denom = y[:, d:d + 1]
    zero_row = denom == 0.0
    r = 1.0 / jnp.where(zero_row, 1.0, denom)
    u = jnp.where(zero_row, 1.0 / n, 0.0)
    return r, u, zero_row


def _project(x, w_ref, asrc_ref, adst_ref, h_ref, ss_ref, sd_ref, n_heads):
    for head in range(n_heads):
        hh = jnp.dot(x, w_ref[head], preferred_element_type=jnp.float32)
        t = jnp.tanh(hh)
        h_ref[head] = hh
        ss_ref[head] = jax.lax.dot_general(
            t, asrc_ref[head], (((1,), (0,)), ((), ())),
            preferred_element_type=jnp.float32)
        sd_ref[head] = jax.lax.dot_general(
            adst_ref[head], t, (((0,), (1,)), ((), ())),
            preferred_element_type=jnp.float32)


def _layer0_kernel(x_ref, w0_ref, as0_ref, ad0_ref, adj_ref, b_ref,
                   w2_ref, as2_ref, ad2_ref,
                   adji_ref, h2_ref, ss2_ref, sd2_ref,
                   h0s, ss0s, sd0s,
                   *, n_heads, d_out, n_heads2, block_rows):
    rr = pl.program_id(1)

    @pl.when(rr == 0)
    def _proj_phase():
        _project(x_ref[0], w0_ref, as0_ref, ad0_ref, h0s, ss0s, sd0s, n_heads)

    @pl.when(rr > 0)
    def _attn_phase():
        adj = adj_ref[0]
        n = adj.shape[1]
        row0 = (rr - 1) * block_rows
        cols = []
        for head in range(n_heads):
            ep, eq, esd, esd2 = _exp_factors(
                ss0s[head, pl.ds(row0, block_rows)], sd0s[head])
            e = jnp.maximum(ep * esd, eq * esd2) * adj
            h = h0s[head]
            y = jnp.dot(e, _ones_col(h), preferred_element_type=jnp.float32)
            r, u, zero_row = _norm_cols(y, d_out, n)
            hm = jnp.sum(h, axis=0, keepdims=True) / n
            xh = jnp.where(zero_row, hm, y[:, :d_out] * r)
            cols.append(xh + b_ref[0])
        x1 = jnp.concatenate(cols, axis=1)
        adji_ref[0] = adj.astype(jnp.int8)
        for oh in range(n_heads2):
            hh = jnp.dot(x1, w2_ref[oh], preferred_element_type=jnp.float32)
            t = jnp.tanh(hh)
            h2_ref[0, oh] = hh
            ss2_ref[0, oh] = jax.lax.dot_general(
                t, as2_ref[oh], (((1,), (0,)), ((), ())),
                preferred_element_type=jnp.float32)
            sd2_ref[0, oh] = jax.lax.dot_general(
                ad2_ref[oh], t, (((0,), (1,)), ((), ())),
                preferred_element_type=jnp.float32)


def _attn1_kernel(ss_ref, sd_ref, adji_ref, h_ref, b_ref, x_ref, w_ref,
                  *, n_heads, d_out):
    masked = adji_ref[0] == 0
    n = masked.shape[1]
    for head in range(n_heads):
        ep, eq, esd, esd2 = _exp_factors(ss_ref[0, head], sd_ref[0, head])
        e = jnp.where(masked, 0.0, jnp.maximum(ep * esd, eq * esd2))
        h = h_ref[0, head]
        y = jnp.dot(e, _ones_col(h), preferred_element_type=jnp.float32)
        r, u, zero_row = _norm_cols(y, d_out, n)
        w_ref[0, head] = e * r + u
        hm = jnp.sum(h, axis=0, keepdims=True) / n
        xh = jnp.where(zero_row, hm, y[:, :d_out] * r)
        x_ref[0, :, head * d_out:(head + 1) * d_out] = xh + b_ref[0]


def kernel(node_feature, adj, w0, a_src0, a_dst0, b0, w1, a_src1, a_dst1, b1):
    B, N, in_dim = node_feature.shape
    H, _, D = w0.shape
    H2, in2, D2 = w1.shape

    BK0 = 512
    R0 = N // BK0
    prev = lambda rr: jnp.maximum(rr - 1, 0)
    adj_i8, h1, ss1, sd1 = pl.pallas_call(
        functools.partial(_layer0_kernel, n_heads=H, d_out=D, n_heads2=H2,
                          block_rows=BK0),
        grid=(B, R0 + 1),
        in_specs=[
            pl.BlockSpec((1, N, in_dim), lambda bb, rr: (bb, 0, 0)),
            pl.BlockSpec((H, in_dim, D), lambda bb, rr: (0, 0, 0)),
            pl.BlockSpec((H, D, 1), lambda bb, rr: (0, 0, 0)),
            pl.BlockSpec((H, D, 1), lambda bb, rr: (0, 0, 0)),
            pl.BlockSpec((1, BK0, N), lambda bb, rr: (bb, prev(rr), 0)),
            pl.BlockSpec((1, D), lambda bb, rr: (0, 0)),
            pl.BlockSpec((H2, in2, D2), lambda bb, rr: (0, 0, 0)),
            pl.BlockSpec((H2, D2, 1), lambda bb, rr: (0, 0, 0)),
            pl.BlockSpec((H2, D2, 1), lambda bb, rr: (0, 0, 0)),
        ],
        out_specs=[
            pl.BlockSpec((1, BK0, N), lambda bb, rr: (bb, prev(rr), 0)),
            pl.BlockSpec((1, H2, BK0, D2), lambda bb, rr: (bb, 0, prev(rr), 0)),
            pl.BlockSpec((1, H2, BK0, 1), lambda bb, rr: (bb, 0, prev(rr), 0)),
            pl.BlockSpec((1, H2, 1, BK0), lambda bb, rr: (bb, 0, 0, prev(rr))),
        ],
        out_shape=[
            jax.ShapeDtypeStruct((B, N, N), jnp.int8),
            jax.ShapeDtypeStruct((B, H2, N, D2), jnp.float32),
            jax.ShapeDtypeStruct((B, H2, N, 1), jnp.float32),
            jax.ShapeDtypeStruct((B, H2, 1, N), jnp.float32),
        ],
        scratch_shapes=[
            pltpu.VMEM((H, N, D), jnp.float32),
            pltpu.VMEM((H, N, 1), jnp.float32),
            pltpu.VMEM((H, 1, N), jnp.float32),
        ],
        compiler_params=pltpu.CompilerParams(
            dimension_semantics=("parallel", "arbitrary")),
    )(node_feature, w0, a_src0, a_dst0, adj, b0.reshape(1, D),
      w1, a_src1, a_dst1)

    BK1 = 256
    R1 = N // BK1
    x_out, weight = pl.pallas_call(
        functools.partial(_attn1_kernel, n_heads=H2, d_out=D2),
        grid=(B, R1),
        in_specs=[
            pl.BlockSpec((1, H2, BK1, 1), lambda bb, rr: (bb, 0, rr, 0)),
            pl.BlockSpec((1, H2, 1, N), lambda bb, rr: (bb, 0, 0, 0)),
            pl.BlockSpec((1, BK1, N), lambda bb, rr: (bb, rr, 0)),
            pl.BlockSpec((1, H2, N, D2), lambda bb, rr: (bb, 0, 0, 0)),
            pl.BlockSpec((1, D2), lambda bb, rr: (0, 0)),
        ],
        out_specs=[
            pl.BlockSpec((1, BK1, H2 * D2), lambda bb, rr: (bb, rr, 0)),
            pl.BlockSpec((1, H2, BK1, N), lambda bb, rr: (bb, 0, rr, 0)),
        ],
        out_shape=[
            jax.ShapeDtypeStruct((B, N, H2 * D2), jnp.float32),
            jax.ShapeDtypeStruct((B, H2, N, N), jnp.float32),
        ],
        compiler_params=pltpu.CompilerParams(
            dimension_semantics=("parallel", "parallel")),
    )(ss1, sd1, adj_i8, h1, b1.reshape(1, D2))

    return x_out, weight
